# baseline (speedup 1.0000x reference)
import functools

import numpy as np
import jax
import jax.numpy as jnp
from jax import lax
from jax.experimental import pallas as pl
from jax.experimental.pallas import tpu as pltpu
from jax.experimental.pallas import tpu_sc as plsc


def _spiral_index_map(n_circle=3, n_sequence=8, steps=(1, 2, 3)):
    width = 2 * n_circle + 1
    ci = cj = n_circle
    circle_coords = {}
    for k in range(1, n_circle + 1):
        coords = []
        i, j = ci - k, cj
        coords.append((i, j))
        moves = ([(0, 1)] * k + [(1, 0)] * (2 * k) + [(0, -1)] * (2 * k)
                 + [(-1, 0)] * (2 * k) + [(0, 1)] * (k - 1))
        for di, dj in moves:
            i += di
            j += dj
            coords.append((i, j))
        for q, cd in enumerate(coords):
            circle_coords[(k, q)] = cd
    seq_len = 1 + sum(steps)
    idx = np.zeros((n_sequence, seq_len), dtype=np.int32)
    for c in range(n_sequence):
        idx[c, 0] = ci * width + cj
        off = 1
        for k in range(1, n_circle + 1):
            s = steps[k - 1]
            pos = list(range(s * c, s * c + s))
            if c % 2 == 1:
                pos = pos[::-1]
            for q in pos:
                i, j = circle_coords[(k, q)]
                idx[c, off] = i * width + j
                off += 1
    return idx.reshape(-1)


_IDX56 = _spiral_index_map()

_NC, _NS = 2, 16
_NW = _NC * _NS


@functools.cache
def _make_sc_scan(c_int, w, n_band, bs):
    n_seq = _IDX56.shape[0]
    assert c_int == _NW

    mesh = plsc.VectorSubcoreMesh(core_axis_name="c", subcore_axis_name="s",
                                  num_cores=_NC, num_subcores=_NS)

    @functools.partial(
        pl.kernel,
        out_type=jax.ShapeDtypeStruct((c_int, 1, n_seq, n_band, bs),
                                      jnp.float32),
        mesh=mesh,
        scratch_types=[pltpu.SemaphoreType.DMA],
        compiler_params=pltpu.CompilerParams(use_tc_tiling_on_sc=True),
    )
    def sc_scan(xt, out, sem):
        wid = lax.axis_index("s") * _NC + lax.axis_index("c")
        for s in range(n_seq):
            p = int(_IDX56[s])
            pltpu.async_copy(xt.at[wid, p // w, p % w], out.at[wid, 0, s],
                             sem)
        for s in range(n_seq):
            pltpu.make_async_copy(xt.at[wid, 0, 0], out.at[wid, 0, s],
                                  sem).wait()

    return sc_scan


def kernel(x):
    bs, c_int, w, w2, n_band = x.shape
    xt = jnp.transpose(x, (1, 2, 3, 4, 0))
    out_t = _make_sc_scan(c_int, w, n_band, bs)(xt)
    return jnp.transpose(out_t, (4, 0, 1, 2, 3))


# speedup vs baseline: 36.5534x; 36.5534x over previous
import functools

import numpy as np
import jax
import jax.numpy as jnp
from jax import lax
from jax.experimental import pallas as pl
from jax.experimental.pallas import tpu as pltpu
from jax.experimental.pallas import tpu_sc as plsc


def _spiral_index_map(n_circle=3, n_sequence=8, steps=(1, 2, 3)):
    width = 2 * n_circle + 1
    ci = cj = n_circle
    circle_coords = {}
    for k in range(1, n_circle + 1):
        coords = []
        i, j = ci - k, cj
        coords.append((i, j))
        moves = ([(0, 1)] * k + [(1, 0)] * (2 * k) + [(0, -1)] * (2 * k)
                 + [(-1, 0)] * (2 * k) + [(0, 1)] * (k - 1))
        for di, dj in moves:
            i += di
            j += dj
            coords.append((i, j))
        for q, cd in enumerate(coords):
            circle_coords[(k, q)] = cd
    seq_len = 1 + sum(steps)
    idx = np.zeros((n_sequence, seq_len), dtype=np.int32)
    for c in range(n_sequence):
        idx[c, 0] = ci * width + cj
        off = 1
        for k in range(1, n_circle + 1):
            s = steps[k - 1]
            pos = list(range(s * c, s * c + s))
            if c % 2 == 1:
                pos = pos[::-1]
            for q in pos:
                i, j = circle_coords[(k, q)]
                idx[c, off] = i * width + j
                off += 1
    return idx.reshape(-1)


_IDX56 = _spiral_index_map()

_NC, _NS = 2, 16
_NW = _NC * _NS


@functools.cache
def _make_sc_scan(c_int, w, n_band, bs):
    n_seq = _IDX56.shape[0]
    assert c_int == _NW

    mesh = plsc.VectorSubcoreMesh(core_axis_name="c", subcore_axis_name="s",
                                  num_cores=_NC, num_subcores=_NS)

    nbuf = 4
    assert n_seq % nbuf == 0

    @functools.partial(
        pl.kernel,
        out_type=jax.ShapeDtypeStruct((c_int, 1, n_seq, n_band, bs),
                                      jnp.float32),
        mesh=mesh,
        scratch_types=(
            [pltpu.VMEM((n_band, bs), jnp.float32) for _ in range(nbuf)]
            + [pltpu.SemaphoreType.DMA for _ in range(2 * nbuf)]
        ),
        compiler_params=pltpu.CompilerParams(use_tc_tiling_on_sc=True),
    )
    def sc_scan(xt, out, *rest):
        buf = rest[:nbuf]
        isem = rest[nbuf:2 * nbuf]
        osem = rest[2 * nbuf:]
        wid = lax.axis_index("s") * _NC + lax.axis_index("c")

        def start_in(s, k):
            p = int(_IDX56[s])
            pltpu.async_copy(xt.at[wid, p // w, p % w], buf[k], isem[k])

        def wait_in(k):
            pltpu.make_async_copy(xt.at[wid, 0, 0], buf[k], isem[k]).wait()

        def start_out(s, k):
            pltpu.async_copy(buf[k], out.at[wid, 0, s], osem[k])

        def wait_out(k):
            pltpu.make_async_copy(buf[k], out.at[wid, 0, 0], osem[k]).wait()

        for k in range(nbuf):
            start_in(k, k)
        for g in range(0, n_seq, nbuf):
            for k in range(nbuf):
                wait_in(k)
                start_out(g + k, k)
            if g + nbuf < n_seq:
                for k in range(nbuf):
                    wait_out(k)
                    start_in(g + nbuf + k, k)
        for k in range(nbuf):
            wait_out(k)

    return sc_scan


def kernel(x):
    bs, c_int, w, w2, n_band = x.shape
    xt = jnp.transpose(x, (1, 2, 3, 4, 0))
    out_t = _make_sc_scan(c_int, w, n_band, bs)(xt)
    return jnp.transpose(out_t, (4, 0, 1, 2, 3))


# speedup vs baseline: 38.4739x; 1.0525x over previous
import functools

import numpy as np
import jax
import jax.numpy as jnp
from jax import lax
from jax.experimental import pallas as pl
from jax.experimental.pallas import tpu as pltpu
from jax.experimental.pallas import tpu_sc as plsc


def _spiral_index_map(n_circle=3, n_sequence=8, steps=(1, 2, 3)):
    width = 2 * n_circle + 1
    ci = cj = n_circle
    circle_coords = {}
    for k in range(1, n_circle + 1):
        coords = []
        i, j = ci - k, cj
        coords.append((i, j))
        moves = ([(0, 1)] * k + [(1, 0)] * (2 * k) + [(0, -1)] * (2 * k)
                 + [(-1, 0)] * (2 * k) + [(0, 1)] * (k - 1))
        for di, dj in moves:
            i += di
            j += dj
            coords.append((i, j))
        for q, cd in enumerate(coords):
            circle_coords[(k, q)] = cd
    seq_len = 1 + sum(steps)
    idx = np.zeros((n_sequence, seq_len), dtype=np.int32)
    for c in range(n_sequence):
        idx[c, 0] = ci * width + cj
        off = 1
        for k in range(1, n_circle + 1):
            s = steps[k - 1]
            pos = list(range(s * c, s * c + s))
            if c % 2 == 1:
                pos = pos[::-1]
            for q in pos:
                i, j = circle_coords[(k, q)]
                idx[c, off] = i * width + j
                off += 1
    return idx.reshape(-1)


_IDX56 = _spiral_index_map()

_NC, _NS = 2, 16
_NW = _NC * _NS


@functools.cache
def _make_sc_scan(c_int, w, n_band, bs):
    n_seq = _IDX56.shape[0]
    assert c_int == _NW

    mesh = plsc.VectorSubcoreMesh(core_axis_name="c", subcore_axis_name="s",
                                  num_cores=_NC, num_subcores=_NS)

    nbuf = 3
    center = (w // 2) * w + (w // 2)
    seq_center = [s for s in range(n_seq) if int(_IDX56[s]) == center]
    seq_ring = [s for s in range(n_seq) if int(_IDX56[s]) != center]
    assert len(seq_ring) % nbuf == 0

    @functools.partial(
        pl.kernel,
        out_type=jax.ShapeDtypeStruct((c_int, 1, n_seq, n_band, bs),
                                      jnp.float32),
        mesh=mesh,
        scratch_types=(
            [pltpu.VMEM((n_band, bs), jnp.float32) for _ in range(nbuf + 1)]
            + [pltpu.SemaphoreType.DMA for _ in range(2 * nbuf + 2)]
        ),
        compiler_params=pltpu.CompilerParams(use_tc_tiling_on_sc=True),
    )
    def sc_scan(xt, out, *rest):
        buf = rest[:nbuf]
        cbuf = rest[nbuf]
        isem = rest[nbuf + 1:2 * nbuf + 1]
        csem_i, csem_o = rest[2 * nbuf + 1], rest[2 * nbuf + 2]
        osem = rest[2 * nbuf + 3:]
        wid = lax.axis_index("s") * _NC + lax.axis_index("c")

        def start_in(s, k):
            p = int(_IDX56[s])
            pltpu.async_copy(xt.at[wid, p // w, p % w], buf[k], isem[k])

        def wait_in(k):
            pltpu.make_async_copy(xt.at[wid, 0, 0], buf[k], isem[k]).wait()

        def start_out(s, k):
            pltpu.async_copy(buf[k], out.at[wid, 0, s], osem[k])

        def wait_out(k):
            pltpu.make_async_copy(buf[k], out.at[wid, 0, 0], osem[k]).wait()

        pltpu.async_copy(xt.at[wid, w // 2, w // 2], cbuf, csem_i)
        for k in range(nbuf):
            start_in(seq_ring[k], k)
        pltpu.make_async_copy(xt.at[wid, 0, 0], cbuf, csem_i).wait()
        for s in seq_center:
            pltpu.async_copy(cbuf, out.at[wid, 0, s], csem_o)

        nring = len(seq_ring)
        for g in range(0, nring, nbuf):
            for k in range(nbuf):
                wait_in(k)
                start_out(seq_ring[g + k], k)
            if g + nbuf < nring:
                for k in range(nbuf):
                    wait_out(k)
                    start_in(seq_ring[g + nbuf + k], k)
        for k in range(nbuf):
            wait_out(k)
        for s in seq_center:
            pltpu.make_async_copy(cbuf, out.at[wid, 0, 0], csem_o).wait()

    return sc_scan


def kernel(x):
    bs, c_int, w, w2, n_band = x.shape
    xt = jnp.transpose(x, (1, 2, 3, 4, 0))
    out_t = _make_sc_scan(c_int, w, n_band, bs)(xt)
    return jnp.transpose(out_t, (4, 0, 1, 2, 3))


# speedup vs baseline: 38.8648x; 1.0102x over previous
import functools

import numpy as np
import jax
import jax.numpy as jnp
from jax import lax
from jax.experimental import pallas as pl
from jax.experimental.pallas import tpu as pltpu
from jax.experimental.pallas import tpu_sc as plsc


def _spiral_index_map(n_circle=3, n_sequence=8, steps=(1, 2, 3)):
    width = 2 * n_circle + 1
    ci = cj = n_circle
    circle_coords = {}
    for k in range(1, n_circle + 1):
        coords = []
        i, j = ci - k, cj
        coords.append((i, j))
        moves = ([(0, 1)] * k + [(1, 0)] * (2 * k) + [(0, -1)] * (2 * k)
                 + [(-1, 0)] * (2 * k) + [(0, 1)] * (k - 1))
        for di, dj in moves:
            i += di
            j += dj
            coords.append((i, j))
        for q, cd in enumerate(coords):
            circle_coords[(k, q)] = cd
    seq_len = 1 + sum(steps)
    idx = np.zeros((n_sequence, seq_len), dtype=np.int32)
    for c in range(n_sequence):
        idx[c, 0] = ci * width + cj
        off = 1
        for k in range(1, n_circle + 1):
            s = steps[k - 1]
            pos = list(range(s * c, s * c + s))
            if c % 2 == 1:
                pos = pos[::-1]
            for q in pos:
                i, j = circle_coords[(k, q)]
                idx[c, off] = i * width + j
                off += 1
    return idx.reshape(-1)


_IDX56 = _spiral_index_map()

_NC, _NS = 2, 16
_NW = _NC * _NS


@functools.cache
def _make_sc_scan(c_int, w, n_band, bs):
    n_seq = _IDX56.shape[0]
    assert c_int == _NW

    mesh = plsc.VectorSubcoreMesh(core_axis_name="c", subcore_axis_name="s",
                                  num_cores=_NC, num_subcores=_NS)

    nbuf = 4
    center = (w // 2) * w + (w // 2)
    seq_center = [s for s in range(n_seq) if int(_IDX56[s]) == center]
    seq_ring = [s for s in range(n_seq) if int(_IDX56[s]) != center]
    assert len(seq_ring) % nbuf == 0

    @functools.partial(
        pl.kernel,
        out_type=jax.ShapeDtypeStruct((c_int, 1, n_seq, n_band, bs),
                                      jnp.float32),
        mesh=mesh,
        scratch_types=(
            [pltpu.VMEM((n_band, bs), jnp.float32) for _ in range(nbuf + 1)]
            + [pltpu.SemaphoreType.DMA for _ in range(2 * nbuf + 2)]
        ),
        compiler_params=pltpu.CompilerParams(use_tc_tiling_on_sc=True),
    )
    def sc_scan(xt, out, *rest):
        buf = rest[:nbuf]
        cbuf = rest[nbuf]
        isem = rest[nbuf + 1:2 * nbuf + 1]
        csem_i, csem_o = rest[2 * nbuf + 1], rest[2 * nbuf + 2]
        osem = rest[2 * nbuf + 3:]
        wid = lax.axis_index("s") * _NC + lax.axis_index("c")

        def start_in(s, k):
            p = int(_IDX56[s])
            pltpu.async_copy(xt.at[wid, p // w, p % w], buf[k], isem[k])

        def wait_in(k):
            pltpu.make_async_copy(xt.at[wid, 0, 0], buf[k], isem[k]).wait()

        def start_out(s, k):
            pltpu.async_copy(buf[k], out.at[wid, 0, s], osem[k])

        def wait_out(k):
            pltpu.make_async_copy(buf[k], out.at[wid, 0, 0], osem[k]).wait()

        pltpu.async_copy(xt.at[wid, w // 2, w // 2], cbuf, csem_i)
        for k in range(nbuf):
            start_in(seq_ring[k], k)
        pltpu.make_async_copy(xt.at[wid, 0, 0], cbuf, csem_i).wait()
        for s in seq_center:
            pltpu.async_copy(cbuf, out.at[wid, 0, s], csem_o)

        nring = len(seq_ring)
        for g in range(0, nring, nbuf):
            for k in range(nbuf):
                wait_in(k)
                start_out(seq_ring[g + k], k)
            if g + nbuf < nring:
                for k in range(nbuf):
                    wait_out(k)
                    start_in(seq_ring[g + nbuf + k], k)
        for k in range(nbuf):
            wait_out(k)
        for s in seq_center:
            pltpu.make_async_copy(cbuf, out.at[wid, 0, 0], csem_o).wait()

    return sc_scan


def kernel(x):
    bs, c_int, w, w2, n_band = x.shape
    xt = jnp.transpose(x, (1, 2, 3, 4, 0))
    out_t = _make_sc_scan(c_int, w, n_band, bs)(xt)
    return jnp.transpose(out_t, (4, 0, 1, 2, 3))
